# Initial kernel scaffold; baseline (speedup 1.0000x reference)
import jax
import jax.numpy as jnp
from jax.experimental import pallas as pl


def kernel(feat, edge_index, weight, bias):
    raise NotImplementedError("write your pallas kernel here")


# speedup vs baseline: 8.2622x; 8.2622x over previous
import functools

import jax
import jax.numpy as jnp
from jax import lax
from jax.experimental import pallas as pl
from jax.experimental.pallas import tpu as pltpu
from jax.experimental.pallas import tpu_sc as plsc

N = 10000
E = 320000
F = 128
F2 = F // 2
NC = 2
NS = 16
NW = NC * NS
EW = E // NW
CH = 125
KT = (E // NS) // CH
RPT = N // NS
RB = 2000

_SC_PARAMS = dict(
    mesh=plsc.VectorSubcoreMesh(core_axis_name="c", subcore_axis_name="s",
                                num_cores=NC, num_subcores=NS),
    compiler_params=pltpu.CompilerParams(needs_layout_passes=False,
                                         use_tc_tiling_on_sc=False),
)


@functools.cache
def _deg_kernel_fn():
    return functools.partial(
        pl.kernel,
        out_type=jax.ShapeDtypeStruct((NW * N,), jnp.float32),
        scratch_types=[
            pltpu.VMEM((EW,), jnp.int32),
            pltpu.VMEM((N,), jnp.float32),
        ],
        mesh=plsc.VectorSubcoreMesh(core_axis_name="c", subcore_axis_name="s",
                                    num_cores=NC, num_subcores=NS),
        compiler_params=pltpu.CompilerParams(needs_layout_passes=False,
                                             use_tc_tiling_on_sc=False),
    )(_deg_body)


def _deg_body(src_hbm, out_hbm, src_v, hist_v):
    c = lax.axis_index("c")
    s = lax.axis_index("s")
    wid = c * NS + s
    pltpu.sync_copy(src_hbm.at[pl.ds(wid * EW, EW)], src_v)

    zeros16 = jnp.zeros((16,), jnp.float32)

    @pl.loop(0, N // 16)
    def _zero(i):
        hist_v[pl.ds(i * 16, 16)] = zeros16

    ones16 = jnp.ones((16,), jnp.float32)

    @pl.loop(0, EW // 16)
    def _accum(i):
        idx = src_v[pl.ds(i * 16, 16)]
        plsc.addupdate_scatter(hist_v, [idx], ones16)

    pltpu.sync_copy(hist_v, out_hbm.at[pl.ds(wid * N, N)])


def _block_norm(degp_ref):
    deg = jnp.sum(degp_ref[:, 0, 0, :], axis=0)
    return lax.rsqrt(jnp.maximum(deg, 1.0))


def _mm_body(feat_ref, w_ref, degp_ref, hs0_ref, hs1_ref):
    norm = _block_norm(degp_ref)
    h = jnp.dot(feat_ref[...], w_ref[...], preferred_element_type=jnp.float32)
    hs = h * norm[:, None]
    hs0_ref[...] = hs[:, :F2]
    hs1_ref[...] = hs[:, F2:]


def _matmul_norm(feat, weight, degp):
    return pl.pallas_call(
        _mm_body,
        grid=(N // RB,),
        in_specs=[
            pl.BlockSpec((RB, F), lambda i: (i, 0)),
            pl.BlockSpec((F, F), lambda i: (0, 0)),
            pl.BlockSpec((NW, 1, 1, RB), lambda i: (0, i, 0, 0)),
        ],
        out_specs=[
            pl.BlockSpec((RB, F2), lambda i: (i, 0)),
            pl.BlockSpec((RB, F2), lambda i: (i, 0)),
        ],
        out_shape=[
            jax.ShapeDtypeStruct((N, F2), jnp.float32),
            jax.ShapeDtypeStruct((N, F2), jnp.float32),
        ],
    )(feat, weight, degp)


@functools.cache
def _scatter_kernel_fn():
    return functools.partial(
        pl.kernel,
        out_type=[
            jax.ShapeDtypeStruct((N, F2), jnp.float32),
            jax.ShapeDtypeStruct((N, F2), jnp.float32),
        ],
        scratch_types=[
            pltpu.VMEM((KT, CH), jnp.int32),
            pltpu.VMEM((KT, CH), jnp.int32),
            pltpu.VMEM((2, CH, F2), jnp.float32),
            pltpu.VMEM_SHARED((N, F2), jnp.float32),
            pltpu.SemaphoreType.DMA,
            pltpu.SemaphoreType.DMA,
        ],
        mesh=plsc.VectorSubcoreMesh(core_axis_name="c", subcore_axis_name="s",
                                    num_cores=NC, num_subcores=NS),
        compiler_params=pltpu.CompilerParams(needs_layout_passes=False,
                                             use_tc_tiling_on_sc=False),
    )(_scatter_body)


def _scatter_body(hs0_hbm, hs1_hbm, src_hbm, dst_hbm, zeros_hbm,
                  out0_hbm, out1_hbm,
                  src_v, dst_v, rows_v, acc_s, sem0, sem1):
    c = lax.axis_index("c")
    s = lax.axis_index("s")
    sems = (sem0, sem1)

    pltpu.sync_copy(zeros_hbm, acc_s.at[pl.ds(s * RPT, RPT)])
    pltpu.sync_copy(src_hbm.at[pl.ds(s * KT, KT)], src_v)
    pltpu.sync_copy(dst_hbm.at[pl.ds(s * KT, KT)], dst_v)
    plsc.subcore_barrier()

    def _edge_loop(hs_hbm):
        pltpu.async_copy(hs_hbm.at[src_v.at[0]], rows_v.at[0], sems[0])

        @pl.loop(0, KT, step=2)
        def _edges(j0):
            for b in range(2):
                j = j0 + b
                pltpu.make_async_copy(hs_hbm.at[src_v.at[j]], rows_v.at[b],
                                      sems[b]).wait()
                nj = j + 1

                @pl.when(nj < KT)
                def _start_next():
                    pltpu.async_copy(hs_hbm.at[src_v.at[nj]],
                                     rows_v.at[1 - b], sems[1 - b])

                pltpu.sync_copy(rows_v.at[b], acc_s.at[dst_v.at[j]], add=True)

    @pl.when(c == 0)
    def _c0():
        _edge_loop(hs0_hbm)

    @pl.when(c == 1)
    def _c1():
        _edge_loop(hs1_hbm)

    plsc.subcore_barrier()

    @pl.when(c == 0)
    def _w0():
        pltpu.sync_copy(acc_s.at[pl.ds(s * RPT, RPT)],
                        out0_hbm.at[pl.ds(s * RPT, RPT)])

    @pl.when(c == 1)
    def _w1():
        pltpu.sync_copy(acc_s.at[pl.ds(s * RPT, RPT)],
                        out1_hbm.at[pl.ds(s * RPT, RPT)])


def _fin_body(p0_ref, p1_ref, degp_ref, bias_ref, out_ref):
    norm = _block_norm(degp_ref)
    agg = jnp.concatenate([p0_ref[...], p1_ref[...]], axis=1)
    out_ref[...] = agg * norm[:, None] + bias_ref[...]


def _finalize(p0, p1, degp, bias2d):
    return pl.pallas_call(
        _fin_body,
        grid=(N // RB,),
        in_specs=[
            pl.BlockSpec((RB, F2), lambda i: (i, 0)),
            pl.BlockSpec((RB, F2), lambda i: (i, 0)),
            pl.BlockSpec((NW, 1, 1, RB), lambda i: (0, i, 0, 0)),
            pl.BlockSpec((1, F), lambda i: (0, 0)),
        ],
        out_specs=pl.BlockSpec((RB, F), lambda i: (i, 0)),
        out_shape=jax.ShapeDtypeStruct((N, F), jnp.float32),
    )(p0, p1, degp, bias2d)


def kernel(feat, edge_index, weight, bias):
    src = edge_index[0]
    dst = edge_index[1]
    src2d = src.reshape(NS * KT, CH)
    dst2d = dst.reshape(NS * KT, CH)
    zeros = jnp.zeros((RPT, F2), jnp.float32)

    degp = _deg_kernel_fn()(src).reshape(NW, N // RB, 1, RB)
    hs0, hs1 = _matmul_norm(feat, weight, degp)
    agg0, agg1 = _scatter_kernel_fn()(hs0, hs1, src2d, dst2d, zeros)
    return _finalize(agg0, agg1, degp, bias.reshape(1, F))


# speedup vs baseline: 10.4335x; 1.2628x over previous
import functools

import jax
import jax.numpy as jnp
from jax import lax
from jax.experimental import pallas as pl
from jax.experimental.pallas import tpu as pltpu
from jax.experimental.pallas import tpu_sc as plsc

N = 10000
E = 320000
F = 128
F2 = F // 2
NC = 2
NS = 16
NW = NC * NS
EW = E // NW
CH = 125
KT = (E // NS) // CH
RPT = N // NS
RB = 2000

_SC_PARAMS = dict(
    mesh=plsc.VectorSubcoreMesh(core_axis_name="c", subcore_axis_name="s",
                                num_cores=NC, num_subcores=NS),
    compiler_params=pltpu.CompilerParams(needs_layout_passes=False,
                                         use_tc_tiling_on_sc=False),
)


@functools.cache
def _deg_kernel_fn():
    return functools.partial(
        pl.kernel,
        out_type=jax.ShapeDtypeStruct((NW * N,), jnp.float32),
        scratch_types=[
            pltpu.VMEM((EW,), jnp.int32),
            pltpu.VMEM((N,), jnp.float32),
        ],
        mesh=plsc.VectorSubcoreMesh(core_axis_name="c", subcore_axis_name="s",
                                    num_cores=NC, num_subcores=NS),
        compiler_params=pltpu.CompilerParams(needs_layout_passes=False,
                                             use_tc_tiling_on_sc=False),
    )(_deg_body)


def _deg_body(src_hbm, out_hbm, src_v, hist_v):
    c = lax.axis_index("c")
    s = lax.axis_index("s")
    wid = c * NS + s
    pltpu.sync_copy(src_hbm.at[pl.ds(wid * EW, EW)], src_v)

    zeros16 = jnp.zeros((16,), jnp.float32)

    @pl.loop(0, N // 16)
    def _zero(i):
        hist_v[pl.ds(i * 16, 16)] = zeros16

    ones16 = jnp.ones((16,), jnp.float32)

    @pl.loop(0, EW // 16)
    def _accum(i):
        idx = src_v[pl.ds(i * 16, 16)]
        plsc.addupdate_scatter(hist_v, [idx], ones16)

    pltpu.sync_copy(hist_v, out_hbm.at[pl.ds(wid * N, N)])


def _block_norm(degp_ref):
    deg = jnp.sum(degp_ref[:, 0, 0, :], axis=0)
    return lax.rsqrt(jnp.maximum(deg, 1.0))


def _mm_body(feat_ref, w_ref, degp_ref, hs0_ref, hs1_ref):
    norm = _block_norm(degp_ref)
    h = jnp.dot(feat_ref[...], w_ref[...], preferred_element_type=jnp.float32)
    hs = h * norm[:, None]
    hs0_ref[...] = hs[:, :F2]
    hs1_ref[...] = hs[:, F2:]


def _matmul_norm(feat, weight, degp):
    return pl.pallas_call(
        _mm_body,
        grid=(N // RB,),
        in_specs=[
            pl.BlockSpec((RB, F), lambda i: (i, 0)),
            pl.BlockSpec((F, F), lambda i: (0, 0)),
            pl.BlockSpec((NW, 1, 1, RB), lambda i: (0, i, 0, 0)),
        ],
        out_specs=[
            pl.BlockSpec((RB, F2), lambda i: (i, 0)),
            pl.BlockSpec((RB, F2), lambda i: (i, 0)),
        ],
        out_shape=[
            jax.ShapeDtypeStruct((N, F2), jnp.float32),
            jax.ShapeDtypeStruct((N, F2), jnp.float32),
        ],
    )(feat, weight, degp)


@functools.cache
def _scatter_kernel_fn():
    return functools.partial(
        pl.kernel,
        out_type=[
            jax.ShapeDtypeStruct((N, F2), jnp.float32),
            jax.ShapeDtypeStruct((N, F2), jnp.float32),
        ],
        scratch_types=[
            pltpu.VMEM((KT, CH), jnp.int32),
            pltpu.VMEM((KT, CH), jnp.int32),
            pltpu.VMEM((4, CH, F2), jnp.float32),
            pltpu.VMEM_SHARED((N, F2), jnp.float32),
            pltpu.SemaphoreType.DMA,
            pltpu.SemaphoreType.DMA,
            pltpu.SemaphoreType.DMA,
            pltpu.SemaphoreType.DMA,
            pltpu.SemaphoreType.DMA,
            pltpu.SemaphoreType.DMA,
            pltpu.SemaphoreType.DMA,
            pltpu.SemaphoreType.DMA,
        ],
        mesh=plsc.VectorSubcoreMesh(core_axis_name="c", subcore_axis_name="s",
                                    num_cores=NC, num_subcores=NS),
        compiler_params=pltpu.CompilerParams(needs_layout_passes=False,
                                             use_tc_tiling_on_sc=False),
    )(_scatter_body)


def _scatter_body(hs0_hbm, hs1_hbm, src_hbm, dst_hbm, zeros_hbm,
                  out0_hbm, out1_hbm,
                  src_v, dst_v, rows_v, acc_s,
                  gs0, gs1, gs2, gs3, ss0, ss1, ss2, ss3):
    c = lax.axis_index("c")
    s = lax.axis_index("s")
    gsem = (gs0, gs1, gs2, gs3)
    ssem = (ss0, ss1, ss2, ss3)

    pltpu.sync_copy(zeros_hbm, acc_s.at[pl.ds(s * RPT, RPT)])
    pltpu.sync_copy(src_hbm.at[pl.ds(s * KT, KT)], src_v)
    pltpu.sync_copy(dst_hbm.at[pl.ds(s * KT, KT)], dst_v)
    plsc.subcore_barrier()

    def _edge_loop(hs_hbm):
        def _gather(j, b):
            pltpu.async_copy(hs_hbm.at[src_v.at[j]], rows_v.at[b], gsem[b])

        def _scatter(j, b):
            pltpu.async_copy(rows_v.at[b], acc_s.at[dst_v.at[j]], ssem[b],
                             add=True)

        _gather(0, 0)
        _gather(1, 1)

        @pl.loop(0, KT, step=4)
        def _edges(j0):
            for b in range(4):
                j = j0 + b
                pltpu.make_async_copy(hs_hbm.at[src_v.at[j]], rows_v.at[b],
                                      gsem[b]).wait()
                _scatter(j, b)
                b2 = (b + 2) % 4

                @pl.when(j >= 2)
                def _drain_scatter():
                    pltpu.make_async_copy(rows_v.at[b2],
                                          acc_s.at[dst_v.at[j]],
                                          ssem[b2]).wait()

                @pl.when(j + 2 < KT)
                def _start_next():
                    _gather(j + 2, b2)

        pltpu.make_async_copy(rows_v.at[(KT - 2) % 4], acc_s.at[dst_v.at[0]],
                              ssem[(KT - 2) % 4]).wait()
        pltpu.make_async_copy(rows_v.at[(KT - 1) % 4], acc_s.at[dst_v.at[0]],
                              ssem[(KT - 1) % 4]).wait()

    @pl.when(c == 0)
    def _c0():
        _edge_loop(hs0_hbm)

    @pl.when(c == 1)
    def _c1():
        _edge_loop(hs1_hbm)

    plsc.subcore_barrier()

    @pl.when(c == 0)
    def _w0():
        pltpu.sync_copy(acc_s.at[pl.ds(s * RPT, RPT)],
                        out0_hbm.at[pl.ds(s * RPT, RPT)])

    @pl.when(c == 1)
    def _w1():
        pltpu.sync_copy(acc_s.at[pl.ds(s * RPT, RPT)],
                        out1_hbm.at[pl.ds(s * RPT, RPT)])


def _fin_body(p0_ref, p1_ref, degp_ref, bias_ref, out_ref):
    norm = _block_norm(degp_ref)
    agg = jnp.concatenate([p0_ref[...], p1_ref[...]], axis=1)
    out_ref[...] = agg * norm[:, None] + bias_ref[...]


def _finalize(p0, p1, degp, bias2d):
    return pl.pallas_call(
        _fin_body,
        grid=(N // RB,),
        in_specs=[
            pl.BlockSpec((RB, F2), lambda i: (i, 0)),
            pl.BlockSpec((RB, F2), lambda i: (i, 0)),
            pl.BlockSpec((NW, 1, 1, RB), lambda i: (0, i, 0, 0)),
            pl.BlockSpec((1, F), lambda i: (0, 0)),
        ],
        out_specs=pl.BlockSpec((RB, F), lambda i: (i, 0)),
        out_shape=jax.ShapeDtypeStruct((N, F), jnp.float32),
    )(p0, p1, degp, bias2d)


def kernel(feat, edge_index, weight, bias):
    src = edge_index[0]
    dst = edge_index[1]
    src2d = src.reshape(NS * KT, CH)
    dst2d = dst.reshape(NS * KT, CH)
    zeros = jnp.zeros((RPT, F2), jnp.float32)

    degp = _deg_kernel_fn()(src).reshape(NW, N // RB, 1, RB)
    hs0, hs1 = _matmul_norm(feat, weight, degp)
    agg0, agg1 = _scatter_kernel_fn()(hs0, hs1, src2d, dst2d, zeros)
    return _finalize(agg0, agg1, degp, bias.reshape(1, F))


# speedup vs baseline: 11.0106x; 1.0553x over previous
import functools

import jax
import jax.numpy as jnp
from jax import lax
from jax.experimental import pallas as pl
from jax.experimental.pallas import tpu as pltpu
from jax.experimental.pallas import tpu_sc as plsc

N = 10000
E = 320000
F = 128
F2 = F // 2
NC = 2
NS = 16
NW = NC * NS
EW = E // NW
CH = 125
KT = (E // NS) // CH
RPT = N // NS
RB = 2000
NRB = N // RB


@functools.cache
def _deg_kernel_fn():
    return functools.partial(
        pl.kernel,
        out_type=jax.ShapeDtypeStruct((NW * N,), jnp.float32),
        scratch_types=[
            pltpu.VMEM((EW,), jnp.int32),
            pltpu.VMEM((N,), jnp.float32),
        ],
        mesh=plsc.VectorSubcoreMesh(core_axis_name="c", subcore_axis_name="s",
                                    num_cores=NC, num_subcores=NS),
        compiler_params=pltpu.CompilerParams(needs_layout_passes=False,
                                             use_tc_tiling_on_sc=False),
    )(_deg_body)


def _deg_body(src_hbm, out_hbm, src_v, hist_v):
    c = lax.axis_index("c")
    s = lax.axis_index("s")
    wid = c * NS + s
    pltpu.sync_copy(src_hbm.at[pl.ds(wid * EW, EW)], src_v)

    zeros16 = jnp.zeros((16,), jnp.float32)

    @pl.loop(0, N // 16)
    def _zero(i):
        hist_v[pl.ds(i * 16, 16)] = zeros16

    ones16 = jnp.ones((16,), jnp.float32)

    @pl.loop(0, EW // 16)
    def _accum(i):
        idx = src_v[pl.ds(i * 16, 16)]
        plsc.addupdate_scatter(hist_v, [idx], ones16)

    pltpu.sync_copy(hist_v, out_hbm.at[pl.ds(wid * N, N)])


def _mm_body(feat_ref, w_ref, degp_ref, hs0_ref, hs1_ref, norm_ref):
    deg = jnp.sum(degp_ref[:, 0, 0, :], axis=0)
    norm = lax.rsqrt(jnp.maximum(deg, 1.0))
    h = jnp.dot(feat_ref[...], w_ref[...], preferred_element_type=jnp.float32)
    hs = h * norm[:, None]
    hs0_ref[...] = hs[:, :F2]
    hs1_ref[...] = hs[:, F2:]
    norm_ref[...] = norm.reshape(1, 1, RB)


def _matmul_norm(feat, weight, degp):
    return pl.pallas_call(
        _mm_body,
        grid=(NRB,),
        in_specs=[
            pl.BlockSpec((RB, F), lambda i: (i, 0)),
            pl.BlockSpec((F, F), lambda i: (0, 0)),
            pl.BlockSpec((NW, 1, 1, RB), lambda i: (0, i, 0, 0)),
        ],
        out_specs=[
            pl.BlockSpec((RB, F2), lambda i: (i, 0)),
            pl.BlockSpec((RB, F2), lambda i: (i, 0)),
            pl.BlockSpec((1, 1, RB), lambda i: (i, 0, 0)),
        ],
        out_shape=[
            jax.ShapeDtypeStruct((N, F2), jnp.float32),
            jax.ShapeDtypeStruct((N, F2), jnp.float32),
            jax.ShapeDtypeStruct((NRB, 1, RB), jnp.float32),
        ],
    )(feat, weight, degp)


@functools.cache
def _scatter_kernel_fn():
    return functools.partial(
        pl.kernel,
        out_type=jax.ShapeDtypeStruct((N, F), jnp.float32),
        scratch_types=[
            pltpu.VMEM((KT, CH), jnp.int32),
            pltpu.VMEM((KT, CH), jnp.int32),
            pltpu.VMEM((4, CH, F2), jnp.float32),
            pltpu.VMEM((N,), jnp.float32),
            pltpu.VMEM((F2,), jnp.float32),
            pltpu.VMEM_SHARED((N, F2), jnp.float32),
            pltpu.SemaphoreType.DMA,
            pltpu.SemaphoreType.DMA,
            pltpu.SemaphoreType.DMA,
            pltpu.SemaphoreType.DMA,
            pltpu.SemaphoreType.DMA,
            pltpu.SemaphoreType.DMA,
            pltpu.SemaphoreType.DMA,
            pltpu.SemaphoreType.DMA,
        ],
        mesh=plsc.VectorSubcoreMesh(core_axis_name="c", subcore_axis_name="s",
                                    num_cores=NC, num_subcores=NS),
        compiler_params=pltpu.CompilerParams(needs_layout_passes=False,
                                             use_tc_tiling_on_sc=False),
    )(_scatter_body)


def _scatter_body(hs0_hbm, hs1_hbm, src_hbm, dst_hbm, zeros_hbm,
                  norm_hbm, bias_hbm, out_hbm,
                  src_v, dst_v, rows_v, norm_v, bias_v, acc_s,
                  gs0, gs1, gs2, gs3, ss0, ss1, ss2, ss3):
    c = lax.axis_index("c")
    s = lax.axis_index("s")
    gsem = (gs0, gs1, gs2, gs3)
    ssem = (ss0, ss1, ss2, ss3)

    pltpu.sync_copy(zeros_hbm, acc_s.at[pl.ds(s * RPT, RPT)])
    pltpu.sync_copy(src_hbm.at[pl.ds(s * KT, KT)], src_v)
    pltpu.sync_copy(dst_hbm.at[pl.ds(s * KT, KT)], dst_v)
    pltpu.sync_copy(norm_hbm, norm_v)
    pltpu.sync_copy(bias_hbm.at[pl.ds(c * F2, F2)], bias_v)
    plsc.subcore_barrier()

    def _edge_loop(hs_hbm):
        def _gather(j, b):
            pltpu.async_copy(hs_hbm.at[src_v.at[j]], rows_v.at[b], gsem[b])

        def _scatter(j, b):
            pltpu.async_copy(rows_v.at[b], acc_s.at[dst_v.at[j]], ssem[b],
                             add=True)

        _gather(0, 0)
        _gather(1, 1)

        @pl.loop(0, KT, step=4)
        def _edges(j0):
            for b in range(4):
                j = j0 + b
                pltpu.make_async_copy(hs_hbm.at[src_v.at[j]], rows_v.at[b],
                                      gsem[b]).wait()
                _scatter(j, b)
                b2 = (b + 2) % 4

                @pl.when(j >= 2)
                def _drain_scatter():
                    pltpu.make_async_copy(rows_v.at[b2],
                                          acc_s.at[dst_v.at[j]],
                                          ssem[b2]).wait()

                @pl.when(j + 2 < KT)
                def _start_next():
                    _gather(j + 2, b2)

        pltpu.make_async_copy(rows_v.at[(KT - 2) % 4], acc_s.at[dst_v.at[0]],
                              ssem[(KT - 2) % 4]).wait()
        pltpu.make_async_copy(rows_v.at[(KT - 1) % 4], acc_s.at[dst_v.at[0]],
                              ssem[(KT - 1) % 4]).wait()

    @pl.when(c == 0)
    def _c0():
        _edge_loop(hs0_hbm)

    @pl.when(c == 1)
    def _c1():
        _edge_loop(hs1_hbm)

    plsc.subcore_barrier()

    r0 = s * RPT
    biases = [bias_v[pl.ds(k * 16, 16)] for k in range(F2 // 16)]
    for p in range(RPT // CH):
        buf = rows_v.at[p % 4]
        base = r0 + p * CH
        pltpu.sync_copy(acc_s.at[pl.ds(base, CH)], buf)

        @pl.loop(0, CH)
        def _rows(r):
            ridx = jnp.zeros((16,), jnp.int32) + (base + r)
            nrm = plsc.load_gather(norm_v, [ridx])
            for k in range(F2 // 16):
                v = buf[r, pl.ds(k * 16, 16)]
                buf[r, pl.ds(k * 16, 16)] = v * nrm + biases[k]

        pltpu.sync_copy(buf, out_hbm.at[pl.ds(base, CH), pl.ds(c * F2, F2)])


def kernel(feat, edge_index, weight, bias):
    src = edge_index[0]
    dst = edge_index[1]
    src2d = src.reshape(NS * KT, CH)
    dst2d = dst.reshape(NS * KT, CH)
    zeros = jnp.zeros((RPT, F2), jnp.float32)

    degp = _deg_kernel_fn()(src).reshape(NW, NRB, 1, RB)
    hs0, hs1, norm3 = _matmul_norm(feat, weight, degp)
    return _scatter_kernel_fn()(hs0, hs1, src2d, dst2d, zeros,
                                norm3.reshape(N), bias)
